# baseline (speedup 1.0000x reference)
import functools

import jax
import jax.numpy as jnp
from jax import lax
from jax.experimental import pallas as pl
from jax.experimental.pallas import tpu as pltpu
from jax.experimental.pallas import tpu_sc as plsc

B = 16
N = 8192
NSAMP = 2048
L = 16
NCHUNK = N // L


def _fps_body(xyz_hbm, out_hbm, x_v, y_v, z_v, dist_v, idx_v):
    nc = lax.axis_size("c")
    b = lax.axis_index("s") * nc + lax.axis_index("c")

    @pl.when(b < B)
    def _():
        pltpu.sync_copy(xyz_hbm.at[pl.ds(b * 3 * N, N)], x_v)
        pltpu.sync_copy(xyz_hbm.at[pl.ds((b * 3 + 1) * N, N)], y_v)
        pltpu.sync_copy(xyz_hbm.at[pl.ds((b * 3 + 2) * N, N)], z_v)

        def init_chunk(j, carry):
            dist_v[pl.ds(j * L, L)] = jnp.full((L,), 1e10, jnp.float32)
            return carry

        lax.fori_loop(0, NCHUNK, init_chunk, 0)

        lanes = lax.iota(jnp.int32, L)

        def fps_round(i, far):
            fvec = jnp.full((L,), far, jnp.int32)
            cxv = plsc.load_gather(x_v, [fvec])
            cyv = plsc.load_gather(y_v, [fvec])
            czv = plsc.load_gather(z_v, [fvec])
            plsc.store_scatter(
                idx_v, [jnp.full((L,), i, jnp.int32)], fvec, mask=lanes == 0
            )

            carry0 = (jnp.full((L,), -1.0, jnp.float32),
                      jnp.zeros((L,), jnp.int32))

            @plsc.parallel_loop(0, NCHUNK, step=1, unroll=8, carry=carry0)
            def chunk(j, carry):
                rmax, ridx = carry
                sl = pl.ds(j * L, L)
                dx = x_v[sl] - cxv
                dy = y_v[sl] - cyv
                dz = z_v[sl] - czv
                d = (dx * dx + dz * dz) + dy * dy
                nd = jnp.minimum(dist_v[sl], d)
                dist_v[sl] = nd
                m = nd > rmax
                rmax = jnp.where(m, nd, rmax)
                ridx = jnp.where(m, jnp.full((L,), j, jnp.int32), ridx)
                return rmax, ridx

            rmax, ridx = chunk
            gmax = jnp.max(rmax)
            gidx = ridx * L + lanes
            cand = jnp.where(rmax == gmax, gidx, jnp.int32(2**30))
            return jnp.min(cand)

        lax.fori_loop(0, NSAMP, fps_round, jnp.int32(0))
        pltpu.sync_copy(idx_v, out_hbm.at[pl.ds(b * NSAMP, NSAMP)])


@jax.jit
def _fps(xyz):
    mesh = plsc.VectorSubcoreMesh(core_axis_name="c", subcore_axis_name="s")
    flat = pl.kernel(
        _fps_body,
        out_type=jax.ShapeDtypeStruct((B * NSAMP,), jnp.int32),
        mesh=mesh,
        compiler_params=pltpu.CompilerParams(needs_layout_passes=False),
        scratch_types=[
            pltpu.VMEM((N,), jnp.float32),
            pltpu.VMEM((N,), jnp.float32),
            pltpu.VMEM((N,), jnp.float32),
            pltpu.VMEM((N,), jnp.float32),
            pltpu.VMEM((NSAMP,), jnp.int32),
        ],
    )(xyz.reshape(B * 3 * N))
    return flat.reshape(B, NSAMP)


def kernel(xyz):
    return _fps(xyz)


# speedup vs baseline: 1.5138x; 1.5138x over previous
import functools

import jax
import jax.numpy as jnp
from jax import lax
from jax.experimental import pallas as pl
from jax.experimental.pallas import tpu as pltpu
from jax.experimental.pallas import tpu_sc as plsc

B = 16
N = 8192
NSAMP = 2048
L = 16
NHALF = N // 2
NCHUNK = NHALF // L


def _fps_body(xyz_hbm, out_hbm, x_v, y_v, z_v, dist_v, idx_v, mail):
    s = lax.axis_index("s")
    c = lax.axis_index("c")
    b = c * 8 + s // 2
    half = s % 2
    base = half * NHALF

    pltpu.sync_copy(xyz_hbm.at[pl.ds(b * 3 * N, N)], x_v)
    pltpu.sync_copy(xyz_hbm.at[pl.ds((b * 3 + 1) * N, N)], y_v)
    pltpu.sync_copy(xyz_hbm.at[pl.ds((b * 3 + 2) * N, N)], z_v)

    def _init(j, carry):
        dist_v[pl.ds(j * L, L)] = jnp.full((L,), 1e10, jnp.float32)
        return carry

    lax.fori_loop(0, NCHUNK, _init, 0)

    for k in range(6):
        mail[k] = 0
    plsc.subcore_barrier()

    lanes = lax.iota(jnp.int32, L)

    def fps_round(i, fvec):
        cxv = plsc.load_gather(x_v, [fvec])
        cyv = plsc.load_gather(y_v, [fvec])
        czv = plsc.load_gather(z_v, [fvec])
        plsc.store_scatter(
            idx_v, [jnp.full((L,), i, jnp.int32)], fvec, mask=lanes == 0
        )

        carry0 = (jnp.full((L,), -1.0, jnp.float32),
                  jnp.zeros((L,), jnp.int32))

        @plsc.parallel_loop(0, NCHUNK, step=1, unroll=8, carry=carry0)
        def chunk(j, carry):
            rmax, ridx = carry
            sl = pl.ds(base + j * L, L)
            dx = x_v[sl] - cxv
            dy = y_v[sl] - cyv
            dz = z_v[sl] - czv
            d = (dx * dx + dz * dz) + dy * dy
            nd = jnp.minimum(dist_v[pl.ds(j * L, L)], d)
            dist_v[pl.ds(j * L, L)] = nd
            m = nd > rmax
            rmax = jnp.where(m, nd, rmax)
            ridx = jnp.where(m, jnp.full((L,), j, jnp.int32), ridx)
            return rmax, ridx

        rmax, ridx = chunk
        gmax = jnp.max(rmax)
        gidx = base + ridx * L + lanes
        lidx = jnp.min(jnp.where(rmax == gmax, gidx, jnp.int32(2**30)))

        mybits = jnp.max(plsc.bitcast(jnp.full((L,), gmax, jnp.float32),
                                      jnp.int32))
        o = (i % 2) * 3
        partner = s ^ 1
        plsc.fetch_and_add(mail.at[o], mybits, subcore_id=partner)
        plsc.fetch_and_add(mail.at[o + 1], lidx, subcore_id=partner)
        plsc.fetch_and_add(mail.at[o + 2], i + 1, subcore_id=partner)

        def _spin_cond(tag):
            return tag != i + 1

        def _spin_body(tag):
            return plsc.fetch_and_add(mail.at[o + 2], 0, subcore_id=s)

        lax.while_loop(_spin_cond, _spin_body, jnp.int32(0))
        pbits = plsc.fetch_and_add(mail.at[o], 0, subcore_id=s)
        pidx = plsc.fetch_and_add(mail.at[o + 1], 0, subcore_id=s)
        plsc.fetch_and_add(mail.at[o], -pbits, subcore_id=s)
        plsc.fetch_and_add(mail.at[o + 1], -pidx, subcore_id=s)
        plsc.fetch_and_add(mail.at[o + 2], -(i + 1), subcore_id=s)
        take = (pbits > mybits) | ((pbits == mybits) & (pidx < lidx))
        far = jnp.where(take, jnp.clip(pidx, 0, N - 1), lidx)
        return jnp.full((L,), far, jnp.int32)

    lax.fori_loop(0, NSAMP, fps_round, jnp.zeros((L,), jnp.int32))

    @pl.when(half == 0)
    def _():
        pltpu.sync_copy(idx_v, out_hbm.at[pl.ds(b * NSAMP, NSAMP)])


@jax.jit
def _fps(xyz):
    mesh = plsc.VectorSubcoreMesh(core_axis_name="c", subcore_axis_name="s")
    flat = pl.kernel(
        _fps_body,
        out_type=jax.ShapeDtypeStruct((B * NSAMP,), jnp.int32),
        mesh=mesh,
        compiler_params=pltpu.CompilerParams(needs_layout_passes=False),
        scratch_types=[
            pltpu.VMEM((N,), jnp.float32),
            pltpu.VMEM((N,), jnp.float32),
            pltpu.VMEM((N,), jnp.float32),
            pltpu.VMEM((NHALF,), jnp.float32),
            pltpu.VMEM((NSAMP,), jnp.int32),
            pltpu.SMEM((6,), jnp.int32),
        ],
    )(xyz.reshape(B * 3 * N))
    return flat.reshape(B, NSAMP)


def kernel(xyz):
    return _fps(xyz)


# speedup vs baseline: 1.6154x; 1.0671x over previous
import functools

import jax
import jax.numpy as jnp
from jax import lax
from jax.experimental import pallas as pl
from jax.experimental.pallas import tpu as pltpu
from jax.experimental.pallas import tpu_sc as plsc

B = 16
N = 8192
NSAMP = 2048
L = 16
NHALF = N // 2
NCHUNK = NHALF // L


def _fps_body(xyz_hbm, out_hbm, x_v, y_v, z_v, dist_v, idx_v, mail):
    s = lax.axis_index("s")
    c = lax.axis_index("c")
    b = c * 8 + s // 2
    half = s % 2
    base = half * NHALF

    pltpu.sync_copy(xyz_hbm.at[pl.ds(b * 3 * N, N)], x_v)
    pltpu.sync_copy(xyz_hbm.at[pl.ds((b * 3 + 1) * N, N)], y_v)
    pltpu.sync_copy(xyz_hbm.at[pl.ds((b * 3 + 2) * N, N)], z_v)

    def _init(j, carry):
        dist_v[pl.ds(j * L, L)] = jnp.full((L,), 1e10, jnp.float32)
        return carry

    lax.fori_loop(0, NCHUNK, _init, 0)

    for k in range(6):
        mail[k] = 0
    plsc.subcore_barrier()

    lanes = lax.iota(jnp.int32, L)

    def fps_round(i, state):
        fvec, pb0, pi0, pb1, pi1 = state
        cxv = plsc.load_gather(x_v, [fvec])
        cyv = plsc.load_gather(y_v, [fvec])
        czv = plsc.load_gather(z_v, [fvec])
        plsc.store_scatter(
            idx_v, [jnp.full((L,), i, jnp.int32)], fvec, mask=lanes == 0
        )

        carry0 = (jnp.full((L,), -1.0, jnp.float32),
                  jnp.zeros((L,), jnp.int32))

        @plsc.parallel_loop(0, NCHUNK, step=1, unroll=8, carry=carry0)
        def chunk(j, carry):
            rmax, ridx = carry
            sl = pl.ds(base + j * L, L)
            dx = x_v[sl] - cxv
            dy = y_v[sl] - cyv
            dz = z_v[sl] - czv
            d = (dx * dx + dz * dz) + dy * dy
            nd = jnp.minimum(dist_v[pl.ds(j * L, L)], d)
            dist_v[pl.ds(j * L, L)] = nd
            m = nd > rmax
            rmax = jnp.where(m, nd, rmax)
            ridx = jnp.where(m, jnp.full((L,), j, jnp.int32), ridx)
            return rmax, ridx

        rmax, ridx = chunk
        gmax = jnp.max(rmax)
        gidx = base + ridx * L + lanes
        lidx = jnp.min(jnp.where(rmax == gmax, gidx, jnp.int32(2**30)))

        mybits = lax.bitcast_convert_type(gmax, jnp.int32)
        even = i % 2 == 0
        o = jnp.where(even, 0, 3)
        partner = s ^ 1
        prevb = jnp.where(even, pb0, pb1)
        previ = jnp.where(even, pi0, pi1)
        prevt = jnp.where(i < 2, 0, i - 1)
        plsc.fetch_and_add(mail.at[o], mybits - prevb, subcore_id=partner)
        plsc.fetch_and_add(mail.at[o + 1], lidx - previ, subcore_id=partner)
        plsc.fetch_and_add(mail.at[o + 2], i + 1 - prevt, subcore_id=partner)

        def _spin_cond(tag):
            return tag != i + 1

        def _spin_body(tag):
            return plsc.fetch_and_add(mail.at[o + 2], 0, subcore_id=s)

        lax.while_loop(_spin_cond, _spin_body, jnp.int32(0))
        pbits = plsc.fetch_and_add(mail.at[o], 0, subcore_id=s)
        pidx = plsc.fetch_and_add(mail.at[o + 1], 0, subcore_id=s)
        take = (pbits > mybits) | ((pbits == mybits) & (pidx < lidx))
        far = jnp.where(take, jnp.clip(pidx, 0, N - 1), lidx)
        fvec = jnp.full((L,), far, jnp.int32)
        pb0 = jnp.where(even, mybits, pb0)
        pi0 = jnp.where(even, lidx, pi0)
        pb1 = jnp.where(even, pb1, mybits)
        pi1 = jnp.where(even, pi1, lidx)
        return fvec, pb0, pi0, pb1, pi1

    z = jnp.int32(0)
    lax.fori_loop(0, NSAMP, fps_round,
                  (jnp.zeros((L,), jnp.int32), z, z, z, z))

    @pl.when(half == 0)
    def _():
        pltpu.sync_copy(idx_v, out_hbm.at[pl.ds(b * NSAMP, NSAMP)])


@jax.jit
def _fps(xyz):
    mesh = plsc.VectorSubcoreMesh(core_axis_name="c", subcore_axis_name="s")
    flat = pl.kernel(
        _fps_body,
        out_type=jax.ShapeDtypeStruct((B * NSAMP,), jnp.int32),
        mesh=mesh,
        compiler_params=pltpu.CompilerParams(needs_layout_passes=False),
        scratch_types=[
            pltpu.VMEM((N,), jnp.float32),
            pltpu.VMEM((N,), jnp.float32),
            pltpu.VMEM((N,), jnp.float32),
            pltpu.VMEM((NHALF,), jnp.float32),
            pltpu.VMEM((NSAMP,), jnp.int32),
            pltpu.SMEM((6,), jnp.int32),
        ],
    )(xyz.reshape(B * 3 * N))
    return flat.reshape(B, NSAMP)


def kernel(xyz):
    return _fps(xyz)


# speedup vs baseline: 1.7030x; 1.0542x over previous
import functools

import jax
import jax.numpy as jnp
from jax import lax
from jax.experimental import pallas as pl
from jax.experimental.pallas import tpu as pltpu
from jax.experimental.pallas import tpu_sc as plsc

B = 16
N = 8192
NSAMP = 2048
L = 16
NHALF = N // 2
NCHUNK = NHALF // L


def _fps_body(xyz_hbm, out_hbm, x_v, y_v, z_v, dist_v, idx_v, mail):
    s = lax.axis_index("s")
    c = lax.axis_index("c")
    b = c * 8 + s // 2
    half = s % 2
    base = half * NHALF

    pltpu.sync_copy(xyz_hbm.at[pl.ds(b * 3 * N, N)], x_v)
    pltpu.sync_copy(xyz_hbm.at[pl.ds((b * 3 + 1) * N, N)], y_v)
    pltpu.sync_copy(xyz_hbm.at[pl.ds((b * 3 + 2) * N, N)], z_v)

    def _init(j, carry):
        dist_v[pl.ds(j * L, L)] = jnp.full((L,), 1e10, jnp.float32)
        return carry

    lax.fori_loop(0, NCHUNK, _init, 0)

    for k in range(4):
        mail[k] = 0
    plsc.subcore_barrier()

    lanes = lax.iota(jnp.int32, L)

    def fps_round(i, state):
        fvec, pb0, pi0, pb1, pi1 = state
        cxv = plsc.load_gather(x_v, [fvec])
        cyv = plsc.load_gather(y_v, [fvec])
        czv = plsc.load_gather(z_v, [fvec])
        plsc.store_scatter(
            idx_v, [jnp.full((L,), i, jnp.int32)], fvec, mask=lanes == 0
        )

        carry0 = (jnp.full((L,), -1.0, jnp.float32),
                  jnp.zeros((L,), jnp.int32))

        @plsc.parallel_loop(0, NCHUNK, step=1, unroll=8, carry=carry0)
        def chunk(j, carry):
            rmax, ridx = carry
            sl = pl.ds(base + j * L, L)
            dx = x_v[sl] - cxv
            dy = y_v[sl] - cyv
            dz = z_v[sl] - czv
            d = (dx * dx + dz * dz) + dy * dy
            nd = jnp.minimum(dist_v[pl.ds(j * L, L)], d)
            dist_v[pl.ds(j * L, L)] = nd
            m = nd > rmax
            rmax = jnp.where(m, nd, rmax)
            ridx = jnp.where(m, jnp.full((L,), j, jnp.int32), ridx)
            return rmax, ridx

        rmax, ridx = chunk
        gmax = jnp.max(rmax)
        gidx = base + ridx * L + lanes
        lidx = jnp.min(jnp.where(rmax == gmax, gidx, jnp.int32(2**30)))

        mybits = lax.bitcast_convert_type(gmax, jnp.int32)
        mypacked = ((i + 1) << 13) | lidx
        even = i % 2 == 0
        o = jnp.where(even, 0, 2)
        partner = s ^ 1
        prevb = jnp.where(even, pb0, pb1)
        prevp = jnp.where(even, pi0, pi1)
        plsc.fetch_and_add(mail.at[o], mybits - prevb, subcore_id=partner)
        plsc.fetch_and_add(mail.at[o + 1], mypacked - prevp,
                           subcore_id=partner)

        def _spin_cond(packed):
            return (packed >> 13) != i + 1

        def _spin_body(packed):
            return plsc.fetch_and_add(mail.at[o + 1], 0, subcore_id=s)

        ppacked = lax.while_loop(_spin_cond, _spin_body, jnp.int32(0))
        pidx = ppacked & (2**13 - 1)
        pbits = plsc.fetch_and_add(mail.at[o], 0, subcore_id=s)
        take = (pbits > mybits) | ((pbits == mybits) & (pidx < lidx))
        far = jnp.where(take, jnp.clip(pidx, 0, N - 1), lidx)
        fvec = jnp.full((L,), far, jnp.int32)
        pb0 = jnp.where(even, mybits, pb0)
        pi0 = jnp.where(even, mypacked, pi0)
        pb1 = jnp.where(even, pb1, mybits)
        pi1 = jnp.where(even, pi1, mypacked)
        return fvec, pb0, pi0, pb1, pi1

    z = jnp.int32(0)
    lax.fori_loop(0, NSAMP, fps_round,
                  (jnp.zeros((L,), jnp.int32), z, z, z, z))

    @pl.when(half == 0)
    def _():
        pltpu.sync_copy(idx_v, out_hbm.at[pl.ds(b * NSAMP, NSAMP)])


@jax.jit
def _fps(xyz):
    mesh = plsc.VectorSubcoreMesh(core_axis_name="c", subcore_axis_name="s")
    flat = pl.kernel(
        _fps_body,
        out_type=jax.ShapeDtypeStruct((B * NSAMP,), jnp.int32),
        mesh=mesh,
        compiler_params=pltpu.CompilerParams(needs_layout_passes=False),
        scratch_types=[
            pltpu.VMEM((N,), jnp.float32),
            pltpu.VMEM((N,), jnp.float32),
            pltpu.VMEM((N,), jnp.float32),
            pltpu.VMEM((NHALF,), jnp.float32),
            pltpu.VMEM((NSAMP,), jnp.int32),
            pltpu.SMEM((4,), jnp.int32),
        ],
    )(xyz.reshape(B * 3 * N))
    return flat.reshape(B, NSAMP)


def kernel(xyz):
    return _fps(xyz)


# speedup vs baseline: 1.7449x; 1.0246x over previous
import functools

import jax
import jax.numpy as jnp
from jax import lax
from jax.experimental import pallas as pl
from jax.experimental.pallas import tpu as pltpu
from jax.experimental.pallas import tpu_sc as plsc

B = 16
N = 8192
NSAMP = 2048
L = 16
NHALF = N // 2
NCHUNK = NHALF // L


def _fps_body(xyz_hbm, out_hbm, x_v, y_v, z_v, dist_v, idx_v, mail):
    s = lax.axis_index("s")
    c = lax.axis_index("c")
    b = c * 8 + s // 2
    half = s % 2
    base = half * NHALF

    pltpu.sync_copy(xyz_hbm.at[pl.ds(b * 3 * N, N)], x_v)
    pltpu.sync_copy(xyz_hbm.at[pl.ds((b * 3 + 1) * N, N)], y_v)
    pltpu.sync_copy(xyz_hbm.at[pl.ds((b * 3 + 2) * N, N)], z_v)

    def _init(j, carry):
        dist_v[pl.ds(j * L, L)] = jnp.full((L,), 1e10, jnp.float32)
        return carry

    lax.fori_loop(0, NCHUNK, _init, 0)

    for k in range(4):
        mail[k] = 0
    plsc.subcore_barrier()

    lanes = lax.iota(jnp.int32, L)

    def fps_round(i, state):
        fvec, pb0, pi0, pb1, pi1 = state
        cxv = plsc.load_gather(x_v, [fvec])
        cyv = plsc.load_gather(y_v, [fvec])
        czv = plsc.load_gather(z_v, [fvec])
        plsc.store_scatter(
            idx_v, [jnp.full((L,), i, jnp.int32)], fvec, mask=lanes == 0
        )

        carry0 = (jnp.full((L,), -1.0, jnp.float32),
                  jnp.zeros((L,), jnp.int32))

        @plsc.parallel_loop(0, NCHUNK, step=1, unroll=16, carry=carry0)
        def chunk(j, carry):
            rmax, ridx = carry
            sl = pl.ds(base + j * L, L)
            dx = x_v[sl] - cxv
            dy = y_v[sl] - cyv
            dz = z_v[sl] - czv
            d = (dx * dx + dz * dz) + dy * dy
            nd = jnp.minimum(dist_v[pl.ds(j * L, L)], d)
            dist_v[pl.ds(j * L, L)] = nd
            m = nd > rmax
            rmax = jnp.where(m, nd, rmax)
            ridx = jnp.where(m, jnp.full((L,), j, jnp.int32), ridx)
            return rmax, ridx

        rmax, ridx = chunk
        gmax = jnp.max(rmax)
        gidx = base + ridx * L + lanes
        lidx = jnp.min(jnp.where(rmax == gmax, gidx, jnp.int32(2**30)))

        mybits = lax.bitcast_convert_type(gmax, jnp.int32)
        mypacked = ((i + 1) << 13) | lidx
        even = i % 2 == 0
        o = jnp.where(even, 0, 2)
        partner = s ^ 1
        prevb = jnp.where(even, pb0, pb1)
        prevp = jnp.where(even, pi0, pi1)
        plsc.fetch_and_add(mail.at[o], mybits - prevb, subcore_id=partner)
        plsc.fetch_and_add(mail.at[o + 1], mypacked - prevp,
                           subcore_id=partner)

        def _spin_cond(packed):
            return (packed >> 13) != i + 1

        def _spin_body(packed):
            return plsc.fetch_and_add(mail.at[o + 1], 0, subcore_id=s)

        ppacked = lax.while_loop(_spin_cond, _spin_body, jnp.int32(0))
        pidx = ppacked & (2**13 - 1)
        pbits = plsc.fetch_and_add(mail.at[o], 0, subcore_id=s)
        take = (pbits > mybits) | ((pbits == mybits) & (pidx < lidx))
        far = jnp.where(take, jnp.clip(pidx, 0, N - 1), lidx)
        fvec = jnp.full((L,), far, jnp.int32)
        pb0 = jnp.where(even, mybits, pb0)
        pi0 = jnp.where(even, mypacked, pi0)
        pb1 = jnp.where(even, pb1, mybits)
        pi1 = jnp.where(even, pi1, mypacked)
        return fvec, pb0, pi0, pb1, pi1

    z = jnp.int32(0)
    lax.fori_loop(0, NSAMP, fps_round,
                  (jnp.zeros((L,), jnp.int32), z, z, z, z))

    @pl.when(half == 0)
    def _():
        pltpu.sync_copy(idx_v, out_hbm.at[pl.ds(b * NSAMP, NSAMP)])


@jax.jit
def _fps(xyz):
    mesh = plsc.VectorSubcoreMesh(core_axis_name="c", subcore_axis_name="s")
    flat = pl.kernel(
        _fps_body,
        out_type=jax.ShapeDtypeStruct((B * NSAMP,), jnp.int32),
        mesh=mesh,
        compiler_params=pltpu.CompilerParams(needs_layout_passes=False),
        scratch_types=[
            pltpu.VMEM((N,), jnp.float32),
            pltpu.VMEM((N,), jnp.float32),
            pltpu.VMEM((N,), jnp.float32),
            pltpu.VMEM((NHALF,), jnp.float32),
            pltpu.VMEM((NSAMP,), jnp.int32),
            pltpu.SMEM((4,), jnp.int32),
        ],
    )(xyz.reshape(B * 3 * N))
    return flat.reshape(B, NSAMP)


def kernel(xyz):
    return _fps(xyz)
